# baseline (speedup 1.0000x reference)
import functools

import jax
import jax.numpy as jnp
from jax import lax
from jax.experimental import pallas as pl
from jax.experimental.pallas import tpu as pltpu
from jax.experimental.pallas import tpu_sc as plsc

N = 320000
D = 128
C = 1000
NC = 2
NS = 16
NW = NC * NS
RPW = N // NW
CH = 400
NCHUNK = RPW // CH
NGRP = CH // 16
ROWW = 2 * D + 16
W = 128
CPAD = 1152
NACC = 17

_mesh = plsc.VectorSubcoreMesh(core_axis_name="c", subcore_axis_name="s")


@functools.partial(
    pl.kernel,
    out_type=jax.ShapeDtypeStruct((NW, CPAD, ROWW), jnp.float32),
    mesh=_mesh,
    scratch_types=[
        pltpu.VMEM((RPW,), jnp.int32),
        pltpu.VMEM((CH, D), jnp.float32),
        pltpu.VMEM((W, ROWW), jnp.float32),
    ],
)
def _sc_accum(x_hbm, y_hbm, out_hbm, y_v, xbuf, win):
    cid = lax.axis_index("c")
    sid = lax.axis_index("s")
    wid = cid * NS + sid

    zv = jnp.zeros((16,), jnp.float32)

    def _zero_win():
        def _zrow(i, _):
            wr = win.at[i]
            for j in range(ROWW // 16):
                wr[pl.ds(16 * j, 16)] = zv
            return 0

        lax.fori_loop(0, W, _zrow, 0)

    _zero_win()
    for w in range(CPAD // W):
        pltpu.sync_copy(win, out_hbm.at[wid, pl.ds(w * W, W), :])

    pltpu.sync_copy(y_hbm.at[pl.ds(wid * RPW, RPW)], y_v)

    def _store_run(widx, accs):
        wr = win.at[widx]
        for j in range(8):
            wr[pl.ds(16 * j, 16)] = accs[j]
        for j in range(8):
            wr[pl.ds(128 + 16 * j, 16)] = accs[8 + j]
        wr[pl.ds(256, 16)] = accs[16]

    def _flush_win(base):
        base = pl.multiple_of(base, 8)
        pltpu.sync_copy(win, out_hbm.at[wid, pl.ds(base, W), :])
        _zero_win()

    zacc = tuple(jnp.zeros((16,), jnp.float32) for _ in range(NACC))
    y0 = y_v[pl.ds(0, 16)][0]
    base0 = jnp.bitwise_and(y0, jnp.int32(~7))
    carry0 = (y0, base0) + zacc
    ones = jnp.ones((16,), jnp.float32)

    def _chunk(k, carry):
        pltpu.sync_copy(x_hbm.at[pl.ds(wid * RPW + k * CH, CH), :], xbuf)

        def _group(g, c):
            yg = y_v[pl.ds(k * CH + g * 16, 16)]
            for ii in range(16):
                yprev, base = c[0], c[1]
                accs = c[2:]
                yi = yg[ii]
                b = yi != yprev

                @pl.when(b)
                def _(yprev=yprev, base=base, yi=yi, accs=accs):
                    _store_run(yprev - base, accs)

                    @pl.when(yi >= base + W)
                    def _():
                        _flush_win(base)

                slide = jnp.logical_and(b, yi >= base + W)
                yprev2 = jnp.where(b, yi, yprev)
                base2 = jnp.where(slide, jnp.bitwise_and(yi, jnp.int32(~7)),
                                  base)
                keep = jnp.where(b, jnp.float32(0), jnp.float32(1))
                xr = xbuf.at[g * 16 + ii]
                upd = []
                for j in range(8):
                    v = xr[pl.ds(16 * j, 16)]
                    upd.append(accs[j] * keep + v)
                    upd.append(accs[8 + j] * keep + v * v)
                cacc = accs[16] * keep + ones
                c = (yprev2, base2) + tuple(upd[0::2] + upd[1::2]) + (cacc,)
            return c

        return lax.fori_loop(0, NGRP, _group, carry)

    carry = lax.fori_loop(0, NCHUNK, _chunk, carry0)

    _store_run(carry[0] - carry[1], carry[2:])
    _flush_win(carry[1])


def _fin_kernel(p_ref, eps_ref, out_ref, acc_ref):
    i = pl.program_id(0)

    @pl.when(i == 0)
    def _():
        acc_ref[...] = jnp.zeros_like(acc_ref)

    acc_ref[...] += p_ref[0]

    @pl.when(i == NW - 1)
    def _():
        p = acc_ref[...]
        s = p[:C, :D]
        ss = p[:C, D:2 * D]
        cnt = p[:C, 2 * D:2 * D + 1]
        denom = jnp.maximum(cnt, 1.0)
        m = s / denom
        sq = jnp.maximum(ss - 2.0 * m * s + cnt * m * m, 0.0)
        stddev = jnp.sqrt(sq / denom)
        out_ref[...] = m + jnp.clip(eps_ref[...], -2.0, 2.0) * stddev


@jax.jit
def kernel(x, y, eps):
    parts = _sc_accum(x, y.astype(jnp.int32))
    return pl.pallas_call(
        _fin_kernel,
        grid=(NW,),
        in_specs=[
            pl.BlockSpec((1, 1008, ROWW), lambda i: (i, 0, 0)),
            pl.BlockSpec((C, D), lambda i: (0, 0)),
        ],
        out_specs=pl.BlockSpec((C, D), lambda i: (0, 0)),
        out_shape=jax.ShapeDtypeStruct((C, D), jnp.float32),
        scratch_shapes=[pltpu.VMEM((1008, ROWW), jnp.float32)],
    )(parts, eps)


# speedup vs baseline: 2.7281x; 2.7281x over previous
import functools

import jax
import jax.numpy as jnp
from jax import lax
from jax.experimental import pallas as pl
from jax.experimental.pallas import tpu as pltpu
from jax.experimental.pallas import tpu_sc as plsc

N = 320000
D = 128
C = 1000
NC = 2
NS = 16
NW = NC * NS
RPW = N // NW
CH = 400
NCHUNK = RPW // CH
NGRP = CH // 16
ROWW = 2 * D + 16
W = 128
CPAD = 1152
NACC = 17

_mesh = plsc.VectorSubcoreMesh(core_axis_name="c", subcore_axis_name="s")


@functools.partial(
    pl.kernel,
    out_type=jax.ShapeDtypeStruct((NW, CPAD, ROWW), jnp.float32),
    mesh=_mesh,
    scratch_types=[
        pltpu.VMEM((RPW,), jnp.int32),
        pltpu.VMEM((CH, D), jnp.float32),
        pltpu.VMEM((W, ROWW), jnp.float32),
        pltpu.VMEM((ROWW,), jnp.float32),
    ],
)
def _sc_accum(x_hbm, y_hbm, out_hbm, y_v, xbuf, win, accbuf):
    cid = lax.axis_index("c")
    sid = lax.axis_index("s")
    wid = cid * NS + sid

    zv = jnp.zeros((16,), jnp.float32)

    def _zero_win():
        def _zrow(i, _):
            wr = win.at[i]
            for j in range(ROWW // 16):
                wr[pl.ds(16 * j, 16)] = zv
            return 0

        lax.fori_loop(0, W, _zrow, 0)

    _zero_win()
    for w in range(CPAD // W):
        pltpu.sync_copy(win, out_hbm.at[wid, pl.ds(w * W, W), :])

    pltpu.sync_copy(y_hbm.at[pl.ds(wid * RPW, RPW)], y_v)

    def _store_run(widx, accs):
        wr = win.at[widx]
        for j in range(8):
            wr[pl.ds(16 * j, 16)] = accs[j]
        for j in range(8):
            wr[pl.ds(128 + 16 * j, 16)] = accs[8 + j]
        wr[pl.ds(256, 16)] = accs[16]

    def _flush_win(base):
        base = pl.multiple_of(base, 8)
        pltpu.sync_copy(win, out_hbm.at[wid, pl.ds(base, W), :])
        _zero_win()

    zacc = tuple(jnp.zeros((16,), jnp.float32) for _ in range(NACC))
    y0 = y_v[pl.ds(0, 16)][0]
    base0 = jnp.bitwise_and(y0, jnp.int32(~7))
    carry0 = (y0, base0) + zacc
    ones = jnp.ones((16,), jnp.float32)

    def _chunk(k, carry):
        pltpu.sync_copy(x_hbm.at[pl.ds(wid * RPW + k * CH, CH), :], xbuf)

        def _group(g, c):
            yg = y_v[pl.ds(k * CH + g * 16, 16)]
            yprev, base = c[0], c[1]
            accs = c[2:]
            uniform = jnp.logical_and(yg[0] == yprev, yg[15] == yprev)

            def _fast(yprev=yprev, base=base, accs=accs):
                s = list(accs)
                for ii in range(16):
                    xr = xbuf.at[g * 16 + ii]
                    for j in range(8):
                        v = xr[pl.ds(16 * j, 16)]
                        s[j] = s[j] + v
                        s[8 + j] = s[8 + j] + v * v
                s[16] = s[16] + 16.0 * ones
                for t in range(NACC):
                    accbuf[pl.ds(16 * t, 16)] = s[t]
                return yprev, base

            def _slow(yprev=yprev, base=base, accs=accs):
                c = (yprev, base) + accs
                for ii in range(16):
                    yprev, base = c[0], c[1]
                    accs = c[2:]
                    yi = yg[ii]
                    b = yi != yprev

                    @pl.when(b)
                    def _(yprev=yprev, base=base, yi=yi, accs=accs):
                        _store_run(yprev - base, accs)

                        @pl.when(yi >= base + W)
                        def _():
                            _flush_win(base)

                    slide = jnp.logical_and(b, yi >= base + W)
                    yprev2 = jnp.where(b, yi, yprev)
                    base2 = jnp.where(
                        slide, jnp.bitwise_and(yi, jnp.int32(~7)), base)
                    keep = jnp.where(b, jnp.float32(0), jnp.float32(1))
                    xr = xbuf.at[g * 16 + ii]
                    upd = []
                    for j in range(8):
                        v = xr[pl.ds(16 * j, 16)]
                        upd.append(accs[j] * keep + v)
                        upd.append(accs[8 + j] * keep + v * v)
                    cacc = accs[16] * keep + ones
                    c = (yprev2, base2) + tuple(
                        upd[0::2] + upd[1::2]) + (cacc,)
                for t in range(NACC):
                    accbuf[pl.ds(16 * t, 16)] = c[2 + t]
                return c[0], c[1]

            yprev2, base2 = lax.cond(uniform, _fast, _slow)
            newaccs = tuple(accbuf[pl.ds(16 * t, 16)] for t in range(NACC))
            return (yprev2, base2) + newaccs

        return lax.fori_loop(0, NGRP, _group, carry)

    carry = lax.fori_loop(0, NCHUNK, _chunk, carry0)

    _store_run(carry[0] - carry[1], carry[2:])
    _flush_win(carry[1])


def _fin_kernel(p_ref, eps_ref, out_ref, acc_ref):
    i = pl.program_id(0)

    @pl.when(i == 0)
    def _():
        acc_ref[...] = jnp.zeros_like(acc_ref)

    acc_ref[...] += p_ref[0]

    @pl.when(i == NW - 1)
    def _():
        p = acc_ref[...]
        s = p[:C, :D]
        ss = p[:C, D:2 * D]
        cnt = p[:C, 2 * D:2 * D + 1]
        denom = jnp.maximum(cnt, 1.0)
        m = s / denom
        sq = jnp.maximum(ss - 2.0 * m * s + cnt * m * m, 0.0)
        stddev = jnp.sqrt(sq / denom)
        out_ref[...] = m + jnp.clip(eps_ref[...], -2.0, 2.0) * stddev


@jax.jit
def kernel(x, y, eps):
    parts = _sc_accum(x, y.astype(jnp.int32))
    return pl.pallas_call(
        _fin_kernel,
        grid=(NW,),
        in_specs=[
            pl.BlockSpec((1, 1008, ROWW), lambda i: (i, 0, 0)),
            pl.BlockSpec((C, D), lambda i: (0, 0)),
        ],
        out_specs=pl.BlockSpec((C, D), lambda i: (0, 0)),
        out_shape=jax.ShapeDtypeStruct((C, D), jnp.float32),
        scratch_shapes=[pltpu.VMEM((1008, ROWW), jnp.float32)],
    )(parts, eps)


# speedup vs baseline: 2.9352x; 1.0759x over previous
import functools

import jax
import jax.numpy as jnp
from jax import lax
from jax.experimental import pallas as pl
from jax.experimental.pallas import tpu as pltpu
from jax.experimental.pallas import tpu_sc as plsc

N = 320000
D = 128
C = 1000
NC = 2
NS = 16
NW = NC * NS
RPW = N // NW
CH = 400
NCHUNK = RPW // CH
NGRP = CH // 16
ROWW = 2 * D + 16
W = 64
CPAD = 1056
NACC = 17
NZERO = 1024 // W

_mesh = plsc.VectorSubcoreMesh(core_axis_name="c", subcore_axis_name="s")


@functools.partial(
    pl.kernel,
    out_type=jax.ShapeDtypeStruct((NW, CPAD, ROWW), jnp.float32),
    mesh=_mesh,
    scratch_types=[
        pltpu.VMEM((CH, D), jnp.float32),
        pltpu.VMEM((CH, D), jnp.float32),
        pltpu.VMEM((CH,), jnp.int32),
        pltpu.VMEM((CH,), jnp.int32),
        pltpu.VMEM((W, ROWW), jnp.float32),
        pltpu.VMEM((ROWW,), jnp.float32),
        pltpu.SemaphoreType.DMA,
        pltpu.SemaphoreType.DMA,
        pltpu.SemaphoreType.DMA,
    ],
)
def _sc_accum(x_hbm, y_hbm, out_hbm, xba, xbb, yba, ybb, win, accbuf,
              sema, semb, semz):
    cid = lax.axis_index("c")
    sid = lax.axis_index("s")
    wid = cid * NS + sid

    zv = jnp.zeros((16,), jnp.float32)
    ones = jnp.ones((16,), jnp.float32)

    def _zero_win():
        def _zrow(i, _):
            wr = win.at[i]
            for j in range(ROWW // 16):
                wr[pl.ds(16 * j, 16)] = zv
            return 0

        lax.fori_loop(0, W, _zrow, 0)

    def _issue(k, xb, yb, sem):
        pltpu.async_copy(x_hbm.at[pl.ds(wid * RPW + k * CH, CH), :], xb, sem)
        pltpu.async_copy(y_hbm.at[pl.ds(wid * RPW + k * CH, CH)], yb, sem)

    def _wait(xb, yb, sem):
        pltpu.make_async_copy(x_hbm.at[pl.ds(0, CH), :], xb, sem).wait()
        pltpu.make_async_copy(y_hbm.at[pl.ds(0, CH)], yb, sem).wait()

    _zero_win()
    _issue(jnp.int32(0), xba, yba, sema)
    for w in range(NZERO):
        pltpu.async_copy(win, out_hbm.at[wid, pl.ds(w * W, W), :], semz)
    _issue(jnp.int32(1), xbb, ybb, semb)
    for w in range(NZERO):
        pltpu.make_async_copy(win, out_hbm.at[wid, pl.ds(w * W, W), :],
                              semz).wait()
    _wait(xba, yba, sema)

    def _store_run(widx, accs):
        wr = win.at[widx]
        for j in range(8):
            wr[pl.ds(16 * j, 16)] = accs[j]
        for j in range(8):
            wr[pl.ds(128 + 16 * j, 16)] = accs[8 + j]
        wr[pl.ds(256, 16)] = accs[16]

    def _flush_win(base):
        base = pl.multiple_of(base, 8)
        pltpu.sync_copy(win, out_hbm.at[wid, pl.ds(base, W), :])
        _zero_win()

    def _process(xb, yb, carry):
        def _group(g, c):
            yg = yb[pl.ds(g * 16, 16)]
            yprev, base = c[0], c[1]
            accs = c[2:]
            uniform = jnp.logical_and(yg[0] == yprev, yg[15] == yprev)

            def _fast(yprev=yprev, base=base, accs=accs):
                s = list(accs)
                for ii in range(16):
                    xr = xb.at[g * 16 + ii]
                    for j in range(8):
                        v = xr[pl.ds(16 * j, 16)]
                        s[j] = s[j] + v
                        s[8 + j] = s[8 + j] + v * v
                s[16] = s[16] + 16.0 * ones
                for t in range(NACC):
                    accbuf[pl.ds(16 * t, 16)] = s[t]
                return yprev, base

            def _slow(yprev=yprev, base=base, accs=accs):
                c = (yprev, base) + accs
                for ii in range(16):
                    yprev, base = c[0], c[1]
                    accs = c[2:]
                    yi = yg[ii]
                    b = yi != yprev

                    @pl.when(b)
                    def _(yprev=yprev, base=base, yi=yi, accs=accs):
                        _store_run(yprev - base, accs)

                        @pl.when(yi >= base + W)
                        def _():
                            _flush_win(base)

                    slide = jnp.logical_and(b, yi >= base + W)
                    yprev2 = jnp.where(b, yi, yprev)
                    base2 = jnp.where(
                        slide, jnp.bitwise_and(yi, jnp.int32(~7)), base)
                    keep = jnp.where(b, jnp.float32(0), jnp.float32(1))
                    xr = xb.at[g * 16 + ii]
                    upd = []
                    for j in range(8):
                        v = xr[pl.ds(16 * j, 16)]
                        upd.append(accs[j] * keep + v)
                        upd.append(accs[8 + j] * keep + v * v)
                    cacc = accs[16] * keep + ones
                    c = (yprev2, base2) + tuple(
                        upd[0::2] + upd[1::2]) + (cacc,)
                for t in range(NACC):
                    accbuf[pl.ds(16 * t, 16)] = c[2 + t]
                return c[0], c[1]

            yprev2, base2 = lax.cond(uniform, _fast, _slow)
            newaccs = tuple(accbuf[pl.ds(16 * t, 16)] for t in range(NACC))
            return (yprev2, base2) + newaccs

        return lax.fori_loop(0, NGRP, _group, carry)

    zacc = tuple(jnp.zeros((16,), jnp.float32) for _ in range(NACC))
    y0 = yba[pl.ds(0, 16)][0]
    base0 = jnp.bitwise_and(y0, jnp.int32(~7))
    carry = _process(xba, yba, (y0, base0) + zacc)

    def _pair(k, carry):
        _wait(xbb, ybb, semb)
        _issue(2 * k + 2, xba, yba, sema)
        carry = _process(xbb, ybb, carry)
        _wait(xba, yba, sema)

        @pl.when(2 * k + 3 < NCHUNK)
        def _(k=k):
            _issue(2 * k + 3, xbb, ybb, semb)

        return _process(xba, yba, carry)

    carry = lax.fori_loop(0, (NCHUNK - 1) // 2, _pair, carry)

    _store_run(carry[0] - carry[1], carry[2:])
    _flush_win(carry[1])


def _fin_kernel(p_ref, eps_ref, out_ref, acc_ref):
    i = pl.program_id(0)

    @pl.when(i == 0)
    def _():
        acc_ref[...] = jnp.zeros_like(acc_ref)

    acc_ref[...] += p_ref[0]

    @pl.when(i == NW - 1)
    def _():
        p = acc_ref[...]
        s = p[:C, :D]
        ss = p[:C, D:2 * D]
        cnt = p[:C, 2 * D:2 * D + 1]
        denom = jnp.maximum(cnt, 1.0)
        m = s / denom
        sq = jnp.maximum(ss - 2.0 * m * s + cnt * m * m, 0.0)
        stddev = jnp.sqrt(sq / denom)
        out_ref[...] = m + jnp.clip(eps_ref[...], -2.0, 2.0) * stddev


@jax.jit
def kernel(x, y, eps):
    parts = _sc_accum(x, y.astype(jnp.int32))
    return pl.pallas_call(
        _fin_kernel,
        grid=(NW,),
        in_specs=[
            pl.BlockSpec((1, 1008, ROWW), lambda i: (i, 0, 0)),
            pl.BlockSpec((C, D), lambda i: (0, 0)),
        ],
        out_specs=pl.BlockSpec((C, D), lambda i: (0, 0)),
        out_shape=jax.ShapeDtypeStruct((C, D), jnp.float32),
        scratch_shapes=[pltpu.VMEM((1008, ROWW), jnp.float32)],
    )(parts, eps)


# speedup vs baseline: 4.3988x; 1.4986x over previous
import functools

import jax
import jax.numpy as jnp
from jax import lax
from jax.experimental import pallas as pl
from jax.experimental.pallas import tpu as pltpu
from jax.experimental.pallas import tpu_sc as plsc

N = 320000
D = 128
C = 1000
NC = 2
NS = 16
NW = NC * NS
RPW = N // NW
CH = 384
NCHUNK = 26
NGRP = CH // 32
TAIL = RPW - NCHUNK * CH
ROWW = 2 * D + 16
W = 64
CPAD = 1056
NACC = 17
NZERO = 1024 // W

_mesh = plsc.VectorSubcoreMesh(core_axis_name="c", subcore_axis_name="s")


@functools.partial(
    pl.kernel,
    out_type=jax.ShapeDtypeStruct((NW, CPAD, ROWW), jnp.float32),
    mesh=_mesh,
    scratch_types=[
        pltpu.VMEM((CH, D), jnp.float32),
        pltpu.VMEM((CH, D), jnp.float32),
        pltpu.VMEM((CH,), jnp.int32),
        pltpu.VMEM((CH,), jnp.int32),
        pltpu.VMEM((W, ROWW), jnp.float32),
        pltpu.VMEM((ROWW,), jnp.float32),
        pltpu.SemaphoreType.DMA,
        pltpu.SemaphoreType.DMA,
        pltpu.SemaphoreType.DMA,
    ],
)
def _sc_accum(x_hbm, y_hbm, out_hbm, xba, xbb, yba, ybb, win, accbuf,
              sema, semb, semz):
    cid = lax.axis_index("c")
    sid = lax.axis_index("s")
    wid = cid * NS + sid

    zv = jnp.zeros((16,), jnp.float32)
    ones = jnp.ones((16,), jnp.float32)

    def _zero_win():
        def _zrow(i, _):
            wr = win.at[i]
            for j in range(ROWW // 16):
                wr[pl.ds(16 * j, 16)] = zv
            return 0

        lax.fori_loop(0, W, _zrow, 0)

    def _issue(k, xb, yb, sem):
        pltpu.async_copy(x_hbm.at[pl.ds(wid * RPW + k * CH, CH), :], xb, sem)
        pltpu.async_copy(y_hbm.at[pl.ds(wid * RPW + k * CH, CH)], yb, sem)

    def _wait(xb, yb, sem):
        pltpu.make_async_copy(x_hbm.at[pl.ds(0, CH), :], xb, sem).wait()
        pltpu.make_async_copy(y_hbm.at[pl.ds(0, CH)], yb, sem).wait()

    _zero_win()
    _issue(jnp.int32(0), xba, yba, sema)
    for w in range(NZERO):
        pltpu.async_copy(win, out_hbm.at[wid, pl.ds(w * W, W), :], semz)
    _issue(jnp.int32(1), xbb, ybb, semb)
    for w in range(NZERO):
        pltpu.make_async_copy(win, out_hbm.at[wid, pl.ds(w * W, W), :],
                              semz).wait()
    _wait(xba, yba, sema)

    def _store_run(widx, accs):
        wr = win.at[widx]
        for j in range(8):
            wr[pl.ds(16 * j, 16)] = accs[j]
        for j in range(8):
            wr[pl.ds(128 + 16 * j, 16)] = accs[8 + j]
        wr[pl.ds(256, 16)] = accs[16]

    def _flush_win(base):
        base = pl.multiple_of(base, 8)
        pltpu.sync_copy(win, out_hbm.at[wid, pl.ds(base, W), :])
        _zero_win()

    def _slow_rows(xb, yb, row0, nrows, c):
        def _row(i, c):
            yprev, base = c[0], c[1]
            accs = c[2:]
            yg = yb[pl.ds(row0 + jnp.bitwise_and(i, jnp.int32(~15)), 16)]
            lane = jnp.bitwise_and(i, jnp.int32(15))
            rot = jnp.bitwise_and(lax.iota(jnp.int32, 16) + lane,
                                  jnp.int32(15))
            yi = yg[rot][0]
            b = yi != yprev

            @pl.when(b)
            def _(yprev=yprev, base=base, yi=yi, accs=accs):
                _store_run(yprev - base, accs)

                @pl.when(yi >= base + W)
                def _():
                    _flush_win(base)

            slide = jnp.logical_and(b, yi >= base + W)
            yprev2 = jnp.where(b, yi, yprev)
            base2 = jnp.where(
                slide, jnp.bitwise_and(yi, jnp.int32(~7)), base)
            keep = jnp.where(b, jnp.float32(0), jnp.float32(1))
            xr = xb.at[row0 + i]
            upd = []
            for j in range(8):
                v = xr[pl.ds(16 * j, 16)]
                upd.append(accs[j] * keep + v)
                upd.append(accs[8 + j] * keep + v * v)
            cacc = accs[16] * keep + ones
            return (yprev2, base2) + tuple(upd[0::2] + upd[1::2]) + (cacc,)

        return lax.fori_loop(0, nrows, _row, c)

    def _process(xb, yb, carry):
        def _group(g, c):
            yg0 = yb[pl.ds(g * 32, 16)]
            yg1 = yb[pl.ds(g * 32 + 16, 16)]
            yprev, base = c[0], c[1]
            accs = c[2:]
            uniform = jnp.logical_and(yg0[0] == yprev, yg1[15] == yprev)

            def _fast(accs=accs, yprev=yprev, base=base):
                s = list(accs)
                for ii in range(32):
                    xr = xb.at[g * 32 + ii]
                    for j in range(8):
                        v = xr[pl.ds(16 * j, 16)]
                        s[j] = s[j] + v
                        s[8 + j] = s[8 + j] + v * v
                s[16] = s[16] + 32.0 * ones
                for t in range(NACC):
                    accbuf[pl.ds(16 * t, 16)] = s[t]
                return yprev, base

            def _slow(accs=accs, yprev=yprev, base=base):
                c2 = _slow_rows(xb, yb, g * 32, 32,
                                (yprev, base) + accs)
                for t in range(NACC):
                    accbuf[pl.ds(16 * t, 16)] = c2[2 + t]
                return c2[0], c2[1]

            yprev2, base2 = lax.cond(uniform, _fast, _slow)
            newaccs = tuple(accbuf[pl.ds(16 * t, 16)] for t in range(NACC))
            return (yprev2, base2) + newaccs

        return lax.fori_loop(0, NGRP, _group, carry)

    zacc = tuple(jnp.zeros((16,), jnp.float32) for _ in range(NACC))
    y0 = yba[pl.ds(0, 16)][0]
    base0 = jnp.bitwise_and(y0, jnp.int32(~7))
    carry = (y0, base0) + zacc

    def _pair(k, carry):
        carry = _process(xba, yba, carry)
        _wait(xbb, ybb, semb)

        @pl.when(2 * k + 2 < NCHUNK)
        def _(k=k):
            _issue(2 * k + 2, xba, yba, sema)

        carry = _process(xbb, ybb, carry)

        @pl.when(2 * k + 3 < NCHUNK)
        def _(k=k):
            _issue(2 * k + 3, xbb, ybb, semb)

        @pl.when(2 * k + 2 < NCHUNK)
        def _(k=k):
            _wait(xba, yba, sema)

        return carry

    carry = lax.fori_loop(0, NCHUNK // 2, _pair, carry)

    pltpu.sync_copy(x_hbm.at[pl.ds(wid * RPW + NCHUNK * CH, TAIL), :],
                    xba.at[pl.ds(0, TAIL), :])
    pltpu.sync_copy(y_hbm.at[pl.ds(wid * RPW + NCHUNK * CH, TAIL)],
                    yba.at[pl.ds(0, TAIL)])
    carry = _slow_rows(xba, yba, 0, TAIL, carry)

    _store_run(carry[0] - carry[1], carry[2:])
    _flush_win(carry[1])


def _fin_kernel(p_ref, eps_ref, out_ref, acc_ref):
    i = pl.program_id(0)

    @pl.when(i == 0)
    def _():
        acc_ref[...] = jnp.zeros_like(acc_ref)

    acc_ref[...] += p_ref[0]

    @pl.when(i == NW - 1)
    def _():
        p = acc_ref[...]
        s = p[:C, :D]
        ss = p[:C, D:2 * D]
        cnt = p[:C, 2 * D:2 * D + 1]
        denom = jnp.maximum(cnt, 1.0)
        m = s / denom
        sq = jnp.maximum(ss - 2.0 * m * s + cnt * m * m, 0.0)
        stddev = jnp.sqrt(sq / denom)
        out_ref[...] = m + jnp.clip(eps_ref[...], -2.0, 2.0) * stddev


@jax.jit
def kernel(x, y, eps):
    parts = _sc_accum(x, y.astype(jnp.int32))
    return pl.pallas_call(
        _fin_kernel,
        grid=(NW,),
        in_specs=[
            pl.BlockSpec((1, 1008, ROWW), lambda i: (i, 0, 0)),
            pl.BlockSpec((C, D), lambda i: (0, 0)),
        ],
        out_specs=pl.BlockSpec((C, D), lambda i: (0, 0)),
        out_shape=jax.ShapeDtypeStruct((C, D), jnp.float32),
        scratch_shapes=[pltpu.VMEM((1008, ROWW), jnp.float32)],
    )(parts, eps)
